# Initial kernel scaffold; baseline (speedup 1.0000x reference)
import jax
import jax.numpy as jnp
from jax.experimental import pallas as pl


def kernel(membrane):
    raise NotImplementedError("write your pallas kernel here")


# speedup vs baseline: 18.0040x; 18.0040x over previous
import functools

import jax
import jax.numpy as jnp
import numpy as np
from jax.experimental import pallas as pl
from jax.experimental.pallas import tpu as pltpu

K_FRAC = 0.1
INT_MIN32 = np.int32(-2147483648)


def _topk_mask_block(x_ref, o_ref, *, k):
    x = x_ref[...]
    b = jax.lax.bitcast_convert_type(x, jnp.int32)
    key = b ^ (jax.lax.shift_right_arithmetic(b, 31) & np.int32(0x7FFFFFFF))

    def count_ge(t):
        return jnp.sum((key >= t).astype(jnp.int32), axis=1, keepdims=True)

    thr = jnp.where(count_ge(jnp.zeros_like(key[:, :1])) >= k,
                    np.int32(0), INT_MIN32)

    def body(i, t):
        cand = t | (np.int32(1) << (np.int32(30) - i))
        return jnp.where(count_ge(cand) >= k, cand, t)

    thr = jax.lax.fori_loop(0, 31, body, thr)
    o_ref[...] = jnp.where(key >= thr, x, np.float32(0.0))


@jax.jit
def kernel(membrane):
    rows, n = membrane.shape
    k = max(1, int(K_FRAC * n))
    block_rows = 8
    grid = (rows // block_rows,)
    return pl.pallas_call(
        functools.partial(_topk_mask_block, k=k),
        grid=grid,
        in_specs=[pl.BlockSpec((block_rows, n), lambda i: (i, 0))],
        out_specs=pl.BlockSpec((block_rows, n), lambda i: (i, 0)),
        out_shape=jax.ShapeDtypeStruct((rows, n), membrane.dtype),
        compiler_params=pltpu.CompilerParams(
            dimension_semantics=("arbitrary",),
        ),
    )(membrane)
